# Initial kernel scaffold; baseline (speedup 1.0000x reference)
import jax
import jax.numpy as jnp
from jax.experimental import pallas as pl


def kernel(h, x, edge_index, edge_attr, norm_edge_weight, batch_idx, p, We_h, be_h, Wc1, bc1, Wc2, bc2, Wf1, bf1, Wf2, bf2, Wm1, bm1, Wm2, bm2, Wm3, bm3, Wg, bg, bn_gamma, bn_beta, Wlp, blp, Wfc2, bfc2):
    raise NotImplementedError("write your pallas kernel here")


# speedup vs baseline: 29.6534x; 29.6534x over previous
import functools

import jax
import jax.numpy as jnp
from jax import lax
from jax.experimental import pallas as pl
from jax.experimental.pallas import tpu as pltpu
from jax.experimental.pallas import tpu_sc as plsc

N = 10000
E = 160000
F = 256
NG = 32
NPAD = 10240
NTRASH = 512
NAGG = NPAD + NTRASH
RB = 1000
WB = 2048
F32 = jnp.float32

_mesh_cache = []


def _mesh():
    if not _mesh_cache:
        _mesh_cache.append(
            plsc.VectorSubcoreMesh(core_axis_name="c", subcore_axis_name="s"))
    return _mesh_cache[0]


def _hh_body(h_ref, w_ref, b_ref, o_ref):
    o_ref[...] = jax.lax.dot_general(
        h_ref[...], w_ref[...], (((1,), (1,)), ((), ())),
        preferred_element_type=F32) + b_ref[...]


def _hh_call(h, We_h, be_row):
    return pl.pallas_call(
        _hh_body,
        grid=(N // RB,),
        in_specs=[
            pl.BlockSpec((RB, F), lambda i: (i, 0)),
            pl.BlockSpec((F, F), lambda i: (0, 0)),
            pl.BlockSpec((1, F), lambda i: (0, 0)),
        ],
        out_specs=pl.BlockSpec((RB, F), lambda i: (i, 0)),
        out_shape=jax.ShapeDtypeStruct((N, F), F32),
    )(h, We_h, be_row)


def _maps_body(xT_ref, Wc1_ref, bc1_ref, Wc2_ref, bc2_ref,
               Wf1_ref, bf1_ref, Wf2_ref, bf2_ref, kap_ref, f_ref):
    xT = xT_ref[...]
    a1 = jnp.maximum(Wc1_ref[...] * xT + bc1_ref[...], 0.0)
    kap = jnp.sum(Wc2_ref[...] * a1, axis=0, keepdims=True) + bc2_ref[...]
    kap_ref[...] = jax.nn.sigmoid(kap)
    for i in range(3):
        ai = jnp.maximum(Wf1_ref[i] * xT + bf1_ref[i], 0.0)
        fi = (jnp.sum(Wf2_ref[i] * ai, axis=0, keepdims=True)
              + bf2_ref[i])
        f_ref[i:i + 1, :] = jax.nn.sigmoid(fi)


def _maps_call(xT, Wc1, bc1r, Wc2, bc2r, Wf1, bf1, Wf2, bf2):
    return pl.pallas_call(
        _maps_body,
        out_shape=(jax.ShapeDtypeStruct((1, N), F32),
                   jax.ShapeDtypeStruct((3, N), F32)),
    )(xT, Wc1, bc1r, Wc2, bc2r, Wf1, bf1, Wf2, bf2)


_EB = 6400


def _wm1_body(eaT_ref, Wm1_ref, b_ref, o_ref):
    k = pl.program_id(0)

    @pl.when(k == 0)
    def _():
        o_ref[...] = b_ref[...]

    o_ref[...] += jax.lax.dot_general(
        Wm1_ref[...], eaT_ref[...], (((1,), (1,)), ((), ())),
        preferred_element_type=F32)

    @pl.when(k == (E // _EB) - 1)
    def _():
        o_ref[...] = jnp.maximum(o_ref[...], 0.0)


def _wm1_call(eaT, Wm1, bm1col):
    return pl.pallas_call(
        _wm1_body,
        grid=(E // _EB,),
        in_specs=[
            pl.BlockSpec((1, _EB), lambda k: (0, k)),
            pl.BlockSpec((64, _EB), lambda k: (0, k)),
            pl.BlockSpec((64, 1), lambda k: (0, 0)),
        ],
        out_specs=pl.BlockSpec((64, 1), lambda k: (0, 0)),
        out_shape=jax.ShapeDtypeStruct((64, 1), F32),
    )(eaT, Wm1, bm1col)


def _w2_body(m1_ref, Wm2_ref, b_ref, o_ref):
    o_ref[...] = jnp.maximum(
        jax.lax.dot_general(Wm2_ref[...], m1_ref[...], (((1,), (0,)), ((), ())),
                            preferred_element_type=F32) + b_ref[...], 0.0)


def _w2_call(m1, Wm2, bm2col):
    return pl.pallas_call(
        _w2_body, out_shape=jax.ShapeDtypeStruct((64, 1), F32),
    )(m1, Wm2, bm2col)


def _wrow_body(w2T_ref, Wm3_ref, b_ref, o_ref):
    o_ref[...] = jax.nn.sigmoid(
        jax.lax.dot_general(w2T_ref[...], Wm3_ref[...], (((1,), (1,)), ((), ())),
                            preferred_element_type=F32) + b_ref[...])


def _wrow_call(w2T, Wm3, bm3T):
    return pl.pallas_call(
        _wrow_body,
        grid=(E // _EB,),
        in_specs=[
            pl.BlockSpec((1, 64), lambda k: (0, 0)),
            pl.BlockSpec((_EB, 64), lambda k: (k, 0)),
            pl.BlockSpec((1, _EB), lambda k: (0, k)),
        ],
        out_specs=pl.BlockSpec((1, _EB), lambda k: (0, k)),
        out_shape=jax.ShapeDtypeStruct((1, E), F32),
    )(w2T, Wm3, bm3T)


def _stats_body(hh_ref, a0_ref, a1_ref, a2_ref, a3_ref, Wg_ref, bg_ref,
                y_ref, s_ref, ss_ref):
    i = pl.program_id(0)
    xin = hh_ref[...] + jnp.concatenate(
        [a0_ref[...], a1_ref[...], a2_ref[...], a3_ref[...]], axis=1)
    y = jax.lax.dot_general(xin, Wg_ref[...], (((1,), (1,)), ((), ())),
                            preferred_element_type=F32) + bg_ref[...]
    y_ref[...] = y

    @pl.when(i == 0)
    def _():
        s_ref[...] = jnp.zeros_like(s_ref)
        ss_ref[...] = jnp.zeros_like(ss_ref)

    s_ref[...] += jnp.sum(y, axis=0, keepdims=True)
    ss_ref[...] += jnp.sum(y * y, axis=0, keepdims=True)


def _stats_call(hh, a0, a1, a2, a3, Wg_l, bg_row):
    return pl.pallas_call(
        _stats_body,
        grid=(N // RB,),
        in_specs=[
            pl.BlockSpec((RB, F), lambda i: (i, 0)),
            pl.BlockSpec((RB, 64), lambda i: (i, 0)),
            pl.BlockSpec((RB, 64), lambda i: (i, 0)),
            pl.BlockSpec((RB, 64), lambda i: (i, 0)),
            pl.BlockSpec((RB, 64), lambda i: (i, 0)),
            pl.BlockSpec((F, F), lambda i: (0, 0)),
            pl.BlockSpec((1, F), lambda i: (0, 0)),
        ],
        out_specs=(pl.BlockSpec((RB, F), lambda i: (i, 0)),
                   pl.BlockSpec((1, F), lambda i: (0, 0)),
                   pl.BlockSpec((1, F), lambda i: (0, 0))),
        out_shape=(jax.ShapeDtypeStruct((N, F), F32),
                   jax.ShapeDtypeStruct((1, F), F32),
                   jax.ShapeDtypeStruct((1, F), F32)),
    )(hh, a0, a1, a2, a3, Wg_l, bg_row)


def _finish_body(y_ref, hprev_ref, s_ref, ss_ref, g_ref, b_ref, o_ref):
    mean = s_ref[...] / N
    var = ss_ref[...] / N - mean * mean
    yn = (y_ref[...] - mean) * jax.lax.rsqrt(var + 1e-5) * g_ref[...] + b_ref[...]
    o_ref[...] = jnp.maximum(yn, 0.0) + hprev_ref[...]


def _finish_call(y, hprev, s, ss, gam_row, bet_row):
    return pl.pallas_call(
        _finish_body,
        grid=(N // RB,),
        in_specs=[
            pl.BlockSpec((RB, F), lambda i: (i, 0)),
            pl.BlockSpec((RB, F), lambda i: (i, 0)),
            pl.BlockSpec((1, F), lambda i: (0, 0)),
            pl.BlockSpec((1, F), lambda i: (0, 0)),
            pl.BlockSpec((1, F), lambda i: (0, 0)),
            pl.BlockSpec((1, F), lambda i: (0, 0)),
        ],
        out_specs=pl.BlockSpec((RB, F), lambda i: (i, 0)),
        out_shape=jax.ShapeDtypeStruct((N, F), F32),
    )(y, hprev, s, ss, gam_row, bet_row)


def _curv_body(kap_ref, gf_ref, dg_ref, gfd_ref, o_ref):
    i = pl.program_id(0)
    j = pl.program_id(1)

    @pl.when(jnp.logical_and(i == 0, j == 0))
    def _():
        o_ref[...] = jnp.zeros_like(o_ref)

    kap = kap_ref[...]
    acc = jnp.float32(0.0)
    for t in range(3):
        gf = gf_ref[t:t + 1, :]
        q = 0.5 * dg_ref[t:t + 1, :] - gfd_ref[t:t + 1, :]
        diff = kap * gf - q
        acc += jnp.sum(jnp.maximum(diff, 0.0))

    @pl.when(j == 0)
    def _():
        o_ref[...] += -3.0 * jnp.sum(kap)

    o_ref[...] += acc


def _curv_call(kap2d, gf, dg, gfd):
    return pl.pallas_call(
        _curv_body,
        grid=(N // RB, NPAD // WB),
        in_specs=[
            pl.BlockSpec((RB, 1), lambda i, j: (i, 0)),
            pl.BlockSpec((3, WB), lambda i, j: (0, j)),
            pl.BlockSpec((3, WB), lambda i, j: (0, j)),
            pl.BlockSpec((3, WB), lambda i, j: (0, j)),
        ],
        out_specs=pl.BlockSpec((1, 1), lambda i, j: (0, 0)),
        out_shape=jax.ShapeDtypeStruct((1, 1), F32),
    )(kap2d, gf, dg, gfd)


def _pool_body(hfin_ref, bidx_ref, Wlp_ref, blp_ref, Wfc2_ref, bfc2_ref,
               o_ref, acc_ref):
    i = pl.program_id(0)

    @pl.when(i == 0)
    def _():
        acc_ref[...] = jnp.zeros_like(acc_ref)

    iot = jax.lax.broadcasted_iota(jnp.int32, (NG, RB), 0).astype(F32)
    ohT = jnp.maximum(1.0 - jnp.abs(bidx_ref[0] - iot), 0.0)
    acc_ref[...] += jax.lax.dot_general(
        ohT, hfin_ref[...], (((1,), (0,)), ((), ())),
        preferred_element_type=F32)

    @pl.when(i == (N // RB) - 1)
    def _():
        ho = jnp.maximum(jax.lax.dot_general(
            acc_ref[...], Wlp_ref[...], (((1,), (1,)), ((), ())),
            preferred_element_type=F32) + blp_ref[...], 0.0)
        o_ref[...] = (jnp.sum(ho * Wfc2_ref[...], axis=1, keepdims=True)
                      + bfc2_ref[...])


def _pool_call(hfin, bidx2d, Wlp, blp_row, Wfc2, bfc2_row):
    return pl.pallas_call(
        _pool_body,
        grid=(N // RB,),
        in_specs=[
            pl.BlockSpec((RB, F), lambda i: (i, 0)),
            pl.BlockSpec((1, 1, RB), lambda i: (i, 0, 0)),
            pl.BlockSpec((2 * F, F), lambda i: (0, 0)),
            pl.BlockSpec((1, 2 * F), lambda i: (0, 0)),
            pl.BlockSpec((1, 2 * F), lambda i: (0, 0)),
            pl.BlockSpec((1, 1), lambda i: (0, 0)),
        ],
        out_specs=pl.BlockSpec((NG, 1), lambda i: (0, 0)),
        out_shape=jax.ShapeDtypeStruct((NG, 1), F32),
        scratch_shapes=[pltpu.VMEM((NG, F), F32)],
    )(hfin, bidx2d, Wlp, blp_row, Wfc2, bfc2_row)


_EPS = E // 16
_CE = 2000
_NS = NPAD // 16


def _curv_stats_kernel(fT, w_e, src_e, dst_e, zpad,
                       gf_out, dg_out, gfd_out,
                       f_loc, dfl, gfl, srcb, dstb, wb, v1, v2,
                       df_s, gf_s, dg_s, gfd_s):
    c = lax.axis_index("c")
    s = lax.axis_index("s")
    nb = s * _NS

    for i in range(3):
        @pl.when((i % 2) == c)
        def _i_stage(i=i):
            pltpu.sync_copy(zpad.at[pl.ds(nb, _NS)], df_s.at[pl.ds(nb, _NS)])
            pltpu.sync_copy(zpad.at[pl.ds(nb, _NS)], gf_s.at[pl.ds(nb, _NS)])
            pltpu.sync_copy(zpad.at[pl.ds(nb, _NS)], dg_s.at[pl.ds(nb, _NS)])
            pltpu.sync_copy(zpad.at[pl.ds(nb, _NS)], gfd_s.at[pl.ds(nb, _NS)])
            pltpu.sync_copy(fT.at[pl.ds(i * NPAD, NPAD)], f_loc)
            plsc.subcore_barrier()

            for k in range(_EPS // _CE):
                off = s * _EPS + k * _CE
                pltpu.sync_copy(src_e.at[pl.ds(off, _CE)], srcb)
                pltpu.sync_copy(dst_e.at[pl.ds(off, _CE)], dstb)
                pltpu.sync_copy(w_e.at[pl.ds(off, _CE)], wb)

                def st1(j, _):
                    s16 = srcb[pl.ds(j * 16, 16)]
                    d16 = dstb[pl.ds(j * 16, 16)]
                    w16 = wb[pl.ds(j * 16, 16)]
                    fd = (plsc.load_gather(f_loc, [d16])
                          - plsc.load_gather(f_loc, [s16]))
                    v1[pl.ds(j * 16, 16)] = w16 * fd
                    v2[pl.ds(j * 16, 16)] = 0.5 * w16 * fd * fd
                    return 0

                lax.fori_loop(0, _CE // 16, st1, 0)
                pltpu.sync_copy(v1, df_s.at[srcb], add=True)
                pltpu.sync_copy(v2, gf_s.at[srcb], add=True)
            plsc.subcore_barrier()

            pltpu.sync_copy(df_s, dfl)
            pltpu.sync_copy(gf_s, gfl)

            for k in range(_EPS // _CE):
                off = s * _EPS + k * _CE
                pltpu.sync_copy(src_e.at[pl.ds(off, _CE)], srcb)
                pltpu.sync_copy(dst_e.at[pl.ds(off, _CE)], dstb)
                pltpu.sync_copy(w_e.at[pl.ds(off, _CE)], wb)

                def st2(j, _):
                    s16 = srcb[pl.ds(j * 16, 16)]
                    d16 = dstb[pl.ds(j * 16, 16)]
                    w16 = wb[pl.ds(j * 16, 16)]
                    fd = (plsc.load_gather(f_loc, [d16])
                          - plsc.load_gather(f_loc, [s16]))
                    gd = (plsc.load_gather(gfl, [d16])
                          - plsc.load_gather(gfl, [s16]))
                    dd = (plsc.load_gather(dfl, [d16])
                          - plsc.load_gather(dfl, [s16]))
                    v1[pl.ds(j * 16, 16)] = w16 * gd
                    v2[pl.ds(j * 16, 16)] = 0.5 * w16 * fd * dd
                    return 0

                lax.fori_loop(0, _CE // 16, st2, 0)
                pltpu.sync_copy(v1, dg_s.at[srcb], add=True)
                pltpu.sync_copy(v2, gfd_s.at[srcb], add=True)
            plsc.subcore_barrier()

            pltpu.sync_copy(gf_s.at[pl.ds(nb, _NS)],
                            gf_out.at[pl.ds(i * NPAD + nb, _NS)])
            pltpu.sync_copy(dg_s.at[pl.ds(nb, _NS)],
                            dg_out.at[pl.ds(i * NPAD + nb, _NS)])
            pltpu.sync_copy(gfd_s.at[pl.ds(nb, _NS)],
                            gfd_out.at[pl.ds(i * NPAD + nb, _NS)])


def _curv_stats_call(fT_pad, weights, src, dst, zpad):
    kfn = functools.partial(
        pl.kernel,
        mesh=_mesh(),
        compiler_params=pltpu.CompilerParams(needs_layout_passes=False),
        out_type=(jax.ShapeDtypeStruct((3 * NPAD,), F32),
                  jax.ShapeDtypeStruct((3 * NPAD,), F32),
                  jax.ShapeDtypeStruct((3 * NPAD,), F32)),
        scratch_types=[
            pltpu.VMEM((NPAD,), F32), pltpu.VMEM((NPAD,), F32),
            pltpu.VMEM((NPAD,), F32),
            pltpu.VMEM((_CE,), jnp.int32), pltpu.VMEM((_CE,), jnp.int32),
            pltpu.VMEM((_CE,), F32),
            pltpu.VMEM((_CE,), F32), pltpu.VMEM((_CE,), F32),
            pltpu.VMEM_SHARED((NPAD,), F32), pltpu.VMEM_SHARED((NPAD,), F32),
            pltpu.VMEM_SHARED((NPAD,), F32), pltpu.VMEM_SHARED((NPAD,), F32),
        ],
    )
    return kfn(_curv_stats_kernel)(fT_pad, weights, src, dst, zpad)


_HI_ONE = 1065353216


def _bc(x):
    return jnp.broadcast_to(jnp.asarray(x, jnp.int32), (16,))


def _count_ge(kb, z1):
    z1v = _bc(z1)

    def body(j, a1):
        v = kb[pl.ds(j * 16, 16)]
        return a1 + (v >= z1v).astype(jnp.int32)
    a1 = lax.fori_loop(0, NPAD // 16, body, jnp.zeros((16,), jnp.int32))
    return jnp.sum(a1, axis=0)


def _count_tie_le(kb, t, m):
    tv, mv = _bc(t), _bc(m)
    ii = lax.iota(jnp.int32, 16)

    def body(j, acc):
        v = kb[pl.ds(j * 16, 16)]
        idx = _bc(j * 16) + ii
        return acc + ((v == tv) & (idx <= mv)).astype(jnp.int32)
    acc = lax.fori_loop(0, NPAD // 16, body, jnp.zeros((16,), jnp.int32))
    return jnp.sum(acc, axis=0)


def _select_rewrite_kernel(rvec16, kap_pad, src_e, dst_e, d12_out,
                           rvb, kapf, kb, rml, srcb, dstb, outb):
    c = lax.axis_index("c")
    pltpu.sync_copy(rvec16, rvb)
    r = jnp.sum((lax.iota(jnp.int32, 16) == _bc(c)).astype(jnp.int32)
                * rvb[...], axis=0)
    _select_core(r, kap_pad, src_e, dst_e, d12_out,
                 kapf, kb, rml, srcb, dstb, outb, c)


def _select_core(r, kap_pad, src_e, dst_e, d12_out,
                 kapf, kb, rml, srcb, dstb, outb, c):
    s = lax.axis_index("s")
    pltpu.sync_copy(kap_pad, kapf)

    def tobits(j, _):
        kb[pl.ds(j * 16, 16)] = plsc.bitcast(kapf[pl.ds(j * 16, 16)], jnp.int32)
        return 0
    lax.fori_loop(0, NPAD // 16, tobits, 0)

    def vsearch(_, carry):
        lo, hi = carry
        mid = (lo + hi + 1) >> 1
        big = (_count_ge(kb, mid) >= r).astype(jnp.int32)
        return lo + big * (mid - lo), hi + (1 - big) * (mid - 1 - hi)
    lo, hi = lax.fori_loop(0, 31, vsearch,
                           (jnp.int32(-1), jnp.int32(_HI_ONE)))
    t = lo
    need = r - _count_ge(kb, t + 1)

    def isearch(_, carry):
        lo2, hi2 = carry
        mid = (lo2 + hi2) >> 1
        ok = (_count_tie_le(kb, t, mid) >= need).astype(jnp.int32)
        return (lo2 + (1 - ok) * (mid + 1 - lo2),
                hi2 + ok * (mid - hi2))
    lo2, hi2 = lax.fori_loop(0, 14, isearch,
                             (jnp.int32(0), jnp.int32(N - 1)))
    m = lo2

    tv, mv = _bc(t), _bc(m)
    rposv = _bc((r > 0).astype(jnp.int32))
    ii = lax.iota(jnp.int32, 16)

    def mkmask(j, _):
        v = kb[pl.ds(j * 16, 16)]
        idx = _bc(j * 16) + ii
        rml[pl.ds(j * 16, 16)] = (
            ((v > tv) | ((v == tv) & (idx <= mv))).astype(jnp.int32) * rposv)
        return 0
    lax.fori_loop(0, NPAD // 16, mkmask, 0)

    for k in range(_EPS // _CE):
        off = s * _EPS + k * _CE
        pltpu.sync_copy(src_e.at[pl.ds(off, _CE)], srcb)
        pltpu.sync_copy(dst_e.at[pl.ds(off, _CE)], dstb)

        def rw(j, _):
            s16 = srcb[pl.ds(j * 16, 16)]
            d16 = dstb[pl.ds(j * 16, 16)]
            drop = (plsc.load_gather(rml, [s16])
                    + plsc.load_gather(rml, [d16])) > 0
            trsh = _bc(NPAD) + ((_bc(off + j * 16) + ii) & _bc(NTRASH - 1))
            outb[pl.ds(j * 16, 16)] = plsc.bitcast(
                jnp.where(drop, trsh, d16), F32)
            return 0
        lax.fori_loop(0, _CE // 16, rw, 0)

        pltpu.sync_copy(outb, d12_out.at[pl.ds(c * E + off, _CE)])


def _select_rewrite_call(rvec16, kap_pad, src, dst):
    kfn = functools.partial(
        pl.kernel,
        mesh=_mesh(),
        compiler_params=pltpu.CompilerParams(needs_layout_passes=False),
        out_type=jax.ShapeDtypeStruct((2 * E,), F32),
        scratch_types=[
            pltpu.VMEM((16,), jnp.int32),
            pltpu.VMEM((NPAD,), F32), pltpu.VMEM((NPAD,), jnp.int32),
            pltpu.VMEM((NPAD,), jnp.int32),
            pltpu.VMEM((_CE,), jnp.int32), pltpu.VMEM((_CE,), jnp.int32),
            pltpu.VMEM((_CE,), F32),
        ],
    )
    return kfn(_select_rewrite_kernel)(rvec16, kap_pad, src, dst)


_CA = 400
_NSA = NAGG // 16


def _agg_kernel(hq0, hq1, hq2, hq3, src_e, dst_e, zrows,
                a0_out, a1_out, a2_out, a3_out, srcb, dstb, rows, sem, agg_s):
    c = lax.axis_index("c")
    s = lax.axis_index("s")
    rb = s * _NSA
    pltpu.sync_copy(zrows.at[pl.ds(rb, _NSA)], agg_s.at[pl.ds(rb, _NSA)])
    plsc.subcore_barrier()

    for q in range(2):
        hq_c0 = (hq0, hq1)[q]
        hq_c1 = (hq2, hq3)[q]
        out_c0 = (a0_out, a1_out)[q]
        out_c1 = (a2_out, a3_out)[q]
        for k in range(_EPS // _CA):
            off = s * _EPS + k * _CA
            pltpu.sync_copy(src_e.at[pl.ds(off, _CA)], srcb)
            pltpu.sync_copy(dst_e.at[pl.ds(off, _CA)], dstb)

            @pl.when(c == 0)
            def _():
                pltpu.async_copy(hq_c0.at[srcb], rows, sem).wait()

            @pl.when(c == 1)
            def _():
                pltpu.async_copy(hq_c1.at[srcb], rows, sem).wait()

            pltpu.sync_copy(rows, agg_s.at[dstb], add=True)

        plsc.subcore_barrier()

        @pl.when(c == 0)
        def _():
            pltpu.sync_copy(agg_s.at[pl.ds(rb, _NSA)],
                            out_c0.at[pl.ds(rb, _NSA)])

        @pl.when(c == 1)
        def _():
            pltpu.sync_copy(agg_s.at[pl.ds(rb, _NSA)],
                            out_c1.at[pl.ds(rb, _NSA)])

        if q == 0:
            pltpu.sync_copy(zrows.at[pl.ds(rb, _NSA)],
                            agg_s.at[pl.ds(rb, _NSA)])
            plsc.subcore_barrier()


def _agg_call(hq0, hq1, hq2, hq3, src, dst_l, zrows):
    kfn = functools.partial(
        pl.kernel,
        mesh=_mesh(),
        compiler_params=pltpu.CompilerParams(use_tc_tiling_on_sc=False),
        out_type=tuple(jax.ShapeDtypeStruct((NAGG, 64), F32)
                       for _ in range(4)),
        scratch_types=[
            pltpu.VMEM((_CA,), jnp.int32), pltpu.VMEM((_CA,), jnp.int32),
            pltpu.VMEM((_CA, 64), F32),
            pltpu.SemaphoreType.DMA,
            pltpu.VMEM_SHARED((NAGG, 64), F32),
        ],
    )
    return kfn(_agg_kernel)(hq0, hq1, hq2, hq3, src, dst_l, zrows)


def kernel(h, x, edge_index, edge_attr, norm_edge_weight, batch_idx, p,
           We_h, be_h, Wc1, bc1, Wc2, bc2, Wf1, bf1, Wf2, bf2,
           Wm1, bm1, Wm2, bm2, Wm3, bm3, Wg, bg, bn_gamma, bn_beta,
           Wlp, blp, Wfc2, bfc2):
    src = edge_index[0]
    dst = edge_index[1]

    hh = _hh_call(h, We_h, be_h.reshape(1, F))
    kapT, fT = _maps_call(x.reshape(1, N), Wc1, bc1.reshape(20, 1),
                          Wc2.reshape(20, 1), bc2.reshape(1, 1), Wf1,
                          bf1.reshape(3, 20, 1), Wf2.reshape(3, 20, 1),
                          bf2.reshape(3, 1, 1))

    m1 = _wm1_call(edge_attr.reshape(1, E), Wm1, bm1.reshape(64, 1))
    w2 = _w2_call(m1, Wm2, bm2.reshape(64, 1))
    weights = _wrow_call(w2.reshape(1, 64), Wm3, bm3.reshape(1, E)).reshape(E)

    fT_pad = jnp.pad(fT, ((0, 0), (0, NPAD - N))).reshape(3 * NPAD)
    zpad = jnp.zeros((NPAD,), F32)
    gf, dg, gfd = _curv_stats_call(fT_pad, weights, src, dst, zpad)
    gf = gf.reshape(3, NPAD)
    dg = dg.reshape(3, NPAD)
    gfd = gfd.reshape(3, NPAD)

    kap2d = kapT.reshape(N, 1)
    curv = _curv_call(kap2d, gf, dg, gfd).reshape(())

    p32 = jnp.asarray(p, jnp.int32)
    r1 = (N * p32 * 1) // 100
    r2 = (N * p32 * 2) // 100
    rvec16 = jnp.zeros((16,), jnp.int32).at[0].set(r1).at[1].set(r2)
    kap_pad = jnp.pad(kapT, ((0, 0), (0, NPAD - N)),
                      constant_values=-1.0).reshape(NPAD)
    d12 = lax.bitcast_convert_type(
        _select_rewrite_call(rvec16, kap_pad, src, dst), jnp.int32)
    dst_by_layer = [dst, d12[:E], d12[E:]]

    zrows = jnp.zeros((NAGG, 64), F32)
    hcur = hh
    for layer in range(3):
        a0, a1, a2, a3 = _agg_call(hcur[:, 0:64], hcur[:, 64:128],
                                   hcur[:, 128:192], hcur[:, 192:256],
                                   src, dst_by_layer[layer], zrows)
        y, sm, ssq = _stats_call(hcur, a0, a1, a2, a3,
                                 Wg[layer], bg[layer].reshape(1, F))
        hcur = _finish_call(y, hcur, sm, ssq,
                            bn_gamma[layer].reshape(1, F),
                            bn_beta[layer].reshape(1, F))

    ho = _pool_call(hcur, batch_idx.reshape(N // RB, 1, RB).astype(F32),
                    Wlp, blp.reshape(1, 2 * F), Wfc2.reshape(1, 2 * F),
                    bfc2.reshape(1, 1))
    return ho, curv


# speedup vs baseline: 35.7369x; 1.2052x over previous
import functools

import jax
import jax.numpy as jnp
from jax import lax
from jax.experimental import pallas as pl
from jax.experimental.pallas import tpu as pltpu
from jax.experimental.pallas import tpu_sc as plsc

N = 10000
E = 160000
F = 256
NG = 32
NPAD = 10240
NTRASH = 512
NAGG = NPAD + NTRASH
RB = 1000
WB = 2048
F32 = jnp.float32

_mesh_cache = []


def _mesh():
    if not _mesh_cache:
        _mesh_cache.append(
            plsc.VectorSubcoreMesh(core_axis_name="c", subcore_axis_name="s"))
    return _mesh_cache[0]


def _hh_body(h_ref, w_ref, b_ref, o_ref):
    o_ref[...] = jax.lax.dot_general(
        h_ref[...], w_ref[...], (((1,), (1,)), ((), ())),
        preferred_element_type=F32) + b_ref[...]


def _hh_call(h, We_h, be_row):
    return pl.pallas_call(
        _hh_body,
        grid=(N // RB,),
        in_specs=[
            pl.BlockSpec((RB, F), lambda i: (i, 0)),
            pl.BlockSpec((F, F), lambda i: (0, 0)),
            pl.BlockSpec((1, F), lambda i: (0, 0)),
        ],
        out_specs=pl.BlockSpec((RB, F), lambda i: (i, 0)),
        out_shape=jax.ShapeDtypeStruct((N, F), F32),
    )(h, We_h, be_row)


def _maps_body(x_ref, Wc1_ref, bc1_ref, W2p_ref, bc2_ref,
               Wf1_ref, bf1_ref, bf2_ref,
               kap_ref, f0_ref, f1_ref, f2_ref):
    xc = x_ref[...]
    a1 = jnp.maximum(xc * Wc1_ref[...] + bc1_ref[...], 0.0)
    kd = jax.lax.dot_general(a1, W2p_ref[...], (((1,), (1,)), ((), ())),
                             preferred_element_type=F32)
    kap_ref[...] = kd[:, 0:1] + bc2_ref[...]
    for i, fr in enumerate((f0_ref, f1_ref, f2_ref)):
        ai = jnp.maximum(xc * Wf1_ref[i] + bf1_ref[i], 0.0)
        fd = jax.lax.dot_general(ai, W2p_ref[...], (((1,), (1,)), ((), ())),
                                 preferred_element_type=F32)
        fr[...] = fd[:, i + 1:i + 2] + bf2_ref[i]


def _maps_call(x2d, Wc1r, bc1r, W2p, bc2r, Wf1r, bf1r, bf2r):
    one = jax.ShapeDtypeStruct((N, 1), F32)
    return pl.pallas_call(
        _maps_body,
        grid=(N // RB,),
        in_specs=[
            pl.BlockSpec((RB, 1), lambda i: (i, 0)),
            pl.BlockSpec((1, 20), lambda i: (0, 0)),
            pl.BlockSpec((1, 20), lambda i: (0, 0)),
            pl.BlockSpec((128, 20), lambda i: (0, 0)),
            pl.BlockSpec((1, 1), lambda i: (0, 0)),
            pl.BlockSpec((3, 1, 20), lambda i: (0, 0, 0)),
            pl.BlockSpec((3, 1, 20), lambda i: (0, 0, 0)),
            pl.BlockSpec((3, 1, 1), lambda i: (0, 0, 0)),
        ],
        out_specs=tuple(pl.BlockSpec((RB, 1), lambda i: (i, 0))
                        for _ in range(4)),
        out_shape=(one, one, one, one),
    )(x2d, Wc1r, bc1r, W2p, bc2r, Wf1r, bf1r, bf2r)


_EB = 6400


def _wm1_body(eaT_ref, Wm1_ref, b_ref, o_ref):
    k = pl.program_id(0)

    @pl.when(k == 0)
    def _():
        o_ref[...] = b_ref[...]

    o_ref[...] += jax.lax.dot_general(
        Wm1_ref[...], eaT_ref[...], (((1,), (1,)), ((), ())),
        preferred_element_type=F32)

    @pl.when(k == (E // _EB) - 1)
    def _():
        o_ref[...] = jnp.maximum(o_ref[...], 0.0)


def _wm1_call(eaT, Wm1, bm1col):
    return pl.pallas_call(
        _wm1_body,
        grid=(E // _EB,),
        in_specs=[
            pl.BlockSpec((1, _EB), lambda k: (0, k)),
            pl.BlockSpec((64, _EB), lambda k: (0, k)),
            pl.BlockSpec((64, 1), lambda k: (0, 0)),
        ],
        out_specs=pl.BlockSpec((64, 1), lambda k: (0, 0)),
        out_shape=jax.ShapeDtypeStruct((64, 1), F32),
    )(eaT, Wm1, bm1col)


def _w2_body(m1_ref, Wm2_ref, b_ref, o_ref):
    o_ref[...] = jnp.maximum(
        jax.lax.dot_general(Wm2_ref[...], m1_ref[...], (((1,), (0,)), ((), ())),
                            preferred_element_type=F32) + b_ref[...], 0.0)


def _w2_call(m1, Wm2, bm2col):
    return pl.pallas_call(
        _w2_body, out_shape=jax.ShapeDtypeStruct((64, 1), F32),
    )(m1, Wm2, bm2col)


def _wrow_body(w2T_ref, Wm3_ref, b_ref, o_ref):
    o_ref[...] = jax.nn.sigmoid(
        jax.lax.dot_general(w2T_ref[...], Wm3_ref[...], (((1,), (1,)), ((), ())),
                            preferred_element_type=F32) + b_ref[...])


def _wrow_call(w2T, Wm3, bm3T):
    return pl.pallas_call(
        _wrow_body,
        grid=(E // _EB,),
        in_specs=[
            pl.BlockSpec((1, 64), lambda k: (0, 0)),
            pl.BlockSpec((_EB, 64), lambda k: (k, 0)),
            pl.BlockSpec((1, _EB), lambda k: (0, k)),
        ],
        out_specs=pl.BlockSpec((1, _EB), lambda k: (0, k)),
        out_shape=jax.ShapeDtypeStruct((1, E), F32),
    )(w2T, Wm3, bm3T)


def _stats_body(hh_ref, a0_ref, a1_ref, a2_ref, a3_ref, Wg_ref, bg_ref,
                y_ref, s_ref, ss_ref):
    i = pl.program_id(0)
    xin = hh_ref[...] + jnp.concatenate(
        [a0_ref[...], a1_ref[...], a2_ref[...], a3_ref[...]], axis=1)
    y = jax.lax.dot_general(xin, Wg_ref[...], (((1,), (1,)), ((), ())),
                            preferred_element_type=F32) + bg_ref[...]
    y_ref[...] = y

    @pl.when(i == 0)
    def _():
        s_ref[...] = jnp.zeros_like(s_ref)
        ss_ref[...] = jnp.zeros_like(ss_ref)

    s_ref[...] += jnp.sum(y, axis=0, keepdims=True)
    ss_ref[...] += jnp.sum(y * y, axis=0, keepdims=True)


def _stats_call(hh, a0, a1, a2, a3, Wg_l, bg_row):
    return pl.pallas_call(
        _stats_body,
        grid=(N // RB,),
        in_specs=[
            pl.BlockSpec((RB, F), lambda i: (i, 0)),
            pl.BlockSpec((RB, 64), lambda i: (i, 0)),
            pl.BlockSpec((RB, 64), lambda i: (i, 0)),
            pl.BlockSpec((RB, 64), lambda i: (i, 0)),
            pl.BlockSpec((RB, 64), lambda i: (i, 0)),
            pl.BlockSpec((F, F), lambda i: (0, 0)),
            pl.BlockSpec((1, F), lambda i: (0, 0)),
        ],
        out_specs=(pl.BlockSpec((RB, F), lambda i: (i, 0)),
                   pl.BlockSpec((1, F), lambda i: (0, 0)),
                   pl.BlockSpec((1, F), lambda i: (0, 0))),
        out_shape=(jax.ShapeDtypeStruct((N, F), F32),
                   jax.ShapeDtypeStruct((1, F), F32),
                   jax.ShapeDtypeStruct((1, F), F32)),
    )(hh, a0, a1, a2, a3, Wg_l, bg_row)


def _finish_body(y_ref, hprev_ref, s_ref, ss_ref, g_ref, b_ref, o_ref):
    mean = s_ref[...] / N
    var = ss_ref[...] / N - mean * mean
    yn = (y_ref[...] - mean) * jax.lax.rsqrt(var + 1e-5) * g_ref[...] + b_ref[...]
    o_ref[...] = jnp.maximum(yn, 0.0) + hprev_ref[...]


def _finish_call(y, hprev, s, ss, gam_row, bet_row):
    return pl.pallas_call(
        _finish_body,
        grid=(N // RB,),
        in_specs=[
            pl.BlockSpec((RB, F), lambda i: (i, 0)),
            pl.BlockSpec((RB, F), lambda i: (i, 0)),
            pl.BlockSpec((1, F), lambda i: (0, 0)),
            pl.BlockSpec((1, F), lambda i: (0, 0)),
            pl.BlockSpec((1, F), lambda i: (0, 0)),
            pl.BlockSpec((1, F), lambda i: (0, 0)),
        ],
        out_specs=pl.BlockSpec((RB, F), lambda i: (i, 0)),
        out_shape=jax.ShapeDtypeStruct((N, F), F32),
    )(y, hprev, s, ss, gam_row, bet_row)


def _curv_body(kap_ref, gf_ref, dg_ref, gfd_ref, o_ref):
    i = pl.program_id(0)
    j = pl.program_id(1)

    @pl.when(jnp.logical_and(i == 0, j == 0))
    def _():
        o_ref[...] = jnp.zeros_like(o_ref)

    kap = kap_ref[...]
    acc = jnp.float32(0.0)
    for t in range(3):
        gf = gf_ref[t:t + 1, :]
        q = 0.5 * dg_ref[t:t + 1, :] - gfd_ref[t:t + 1, :]
        diff = kap * gf - q
        acc += jnp.sum(jnp.maximum(diff, 0.0))

    @pl.when(j == 0)
    def _():
        o_ref[...] += -3.0 * jnp.sum(kap)

    o_ref[...] += acc


def _curv_call(kap2d, gf, dg, gfd):
    return pl.pallas_call(
        _curv_body,
        grid=(N // RB, NPAD // WB),
        in_specs=[
            pl.BlockSpec((RB, 1), lambda i, j: (i, 0)),
            pl.BlockSpec((3, WB), lambda i, j: (0, j)),
            pl.BlockSpec((3, WB), lambda i, j: (0, j)),
            pl.BlockSpec((3, WB), lambda i, j: (0, j)),
        ],
        out_specs=pl.BlockSpec((1, 1), lambda i, j: (0, 0)),
        out_shape=jax.ShapeDtypeStruct((1, 1), F32),
    )(kap2d, gf, dg, gfd)


def _pool_body(hfin_ref, bidx_ref, Wlp_ref, blp_ref, Wfc2_ref, bfc2_ref,
               o_ref, acc_ref):
    i = pl.program_id(0)

    @pl.when(i == 0)
    def _():
        acc_ref[...] = jnp.zeros_like(acc_ref)

    iot = jax.lax.broadcasted_iota(jnp.int32, (NG, RB), 0).astype(F32)
    ohT = jnp.maximum(1.0 - jnp.abs(bidx_ref[0] - iot), 0.0)
    acc_ref[...] += jax.lax.dot_general(
        ohT, hfin_ref[...], (((1,), (0,)), ((), ())),
        preferred_element_type=F32)

    @pl.when(i == (N // RB) - 1)
    def _():
        ho = jnp.maximum(jax.lax.dot_general(
            acc_ref[...], Wlp_ref[...], (((1,), (1,)), ((), ())),
            preferred_element_type=F32) + blp_ref[...], 0.0)
        o_ref[...] = (jnp.sum(ho * Wfc2_ref[...], axis=1, keepdims=True)
                      + bfc2_ref[...])


def _pool_call(hfin, bidx2d, Wlp, blp_row, Wfc2, bfc2_row):
    return pl.pallas_call(
        _pool_body,
        grid=(N // RB,),
        in_specs=[
            pl.BlockSpec((RB, F), lambda i: (i, 0)),
            pl.BlockSpec((1, 1, RB), lambda i: (i, 0, 0)),
            pl.BlockSpec((2 * F, F), lambda i: (0, 0)),
            pl.BlockSpec((1, 2 * F), lambda i: (0, 0)),
            pl.BlockSpec((1, 2 * F), lambda i: (0, 0)),
            pl.BlockSpec((1, 1), lambda i: (0, 0)),
        ],
        out_specs=pl.BlockSpec((NG, 1), lambda i: (0, 0)),
        out_shape=jax.ShapeDtypeStruct((NG, 1), F32),
        scratch_shapes=[pltpu.VMEM((NG, F), F32)],
    )(hfin, bidx2d, Wlp, blp_row, Wfc2, bfc2_row)


_EPS = E // 16
_CE = 2000
_NS = NPAD // 16


def _curv_stats_kernel(fT, w_e, src_e, dst_e, zpad,
                       gf_out, dg_out, gfd_out,
                       f_loc, dfl, gfl, srcb, dstb, wb, v1, v2,
                       df_s, gf_s, dg_s, gfd_s):
    c = lax.axis_index("c")
    s = lax.axis_index("s")
    nb = s * _NS

    for i in range(3):
        @pl.when((i % 2) == c)
        def _i_stage(i=i):
            pltpu.sync_copy(zpad.at[pl.ds(nb, _NS)], df_s.at[pl.ds(nb, _NS)])
            pltpu.sync_copy(zpad.at[pl.ds(nb, _NS)], gf_s.at[pl.ds(nb, _NS)])
            pltpu.sync_copy(zpad.at[pl.ds(nb, _NS)], dg_s.at[pl.ds(nb, _NS)])
            pltpu.sync_copy(zpad.at[pl.ds(nb, _NS)], gfd_s.at[pl.ds(nb, _NS)])
            pltpu.sync_copy(fT.at[pl.ds(i * NPAD, NPAD)], f_loc)
            plsc.subcore_barrier()

            for k in range(_EPS // _CE):
                off = s * _EPS + k * _CE
                pltpu.sync_copy(src_e.at[pl.ds(off, _CE)], srcb)
                pltpu.sync_copy(dst_e.at[pl.ds(off, _CE)], dstb)
                pltpu.sync_copy(w_e.at[pl.ds(off, _CE)], wb)

                def st1(j, _):
                    for u in range(5):
                        o = (j * 5 + u) * 16
                        s16 = srcb[pl.ds(o, 16)]
                        d16 = dstb[pl.ds(o, 16)]
                        w16 = wb[pl.ds(o, 16)]
                        fd = (plsc.load_gather(f_loc, [d16])
                              - plsc.load_gather(f_loc, [s16]))
                        v1[pl.ds(o, 16)] = w16 * fd
                        v2[pl.ds(o, 16)] = 0.5 * w16 * fd * fd
                    return 0

                lax.fori_loop(0, _CE // 80, st1, 0)
                pltpu.sync_copy(v1, df_s.at[srcb], add=True)
                pltpu.sync_copy(v2, gf_s.at[srcb], add=True)
            plsc.subcore_barrier()

            pltpu.sync_copy(df_s, dfl)
            pltpu.sync_copy(gf_s, gfl)

            for k in range(_EPS // _CE):
                off = s * _EPS + k * _CE
                pltpu.sync_copy(src_e.at[pl.ds(off, _CE)], srcb)
                pltpu.sync_copy(dst_e.at[pl.ds(off, _CE)], dstb)
                pltpu.sync_copy(w_e.at[pl.ds(off, _CE)], wb)

                def st2(j, _):
                    for u in range(5):
                        o = (j * 5 + u) * 16
                        s16 = srcb[pl.ds(o, 16)]
                        d16 = dstb[pl.ds(o, 16)]
                        w16 = wb[pl.ds(o, 16)]
                        fd = (plsc.load_gather(f_loc, [d16])
                              - plsc.load_gather(f_loc, [s16]))
                        gd = (plsc.load_gather(gfl, [d16])
                              - plsc.load_gather(gfl, [s16]))
                        dd = (plsc.load_gather(dfl, [d16])
                              - plsc.load_gather(dfl, [s16]))
                        v1[pl.ds(o, 16)] = w16 * gd
                        v2[pl.ds(o, 16)] = 0.5 * w16 * fd * dd
                    return 0

                lax.fori_loop(0, _CE // 80, st2, 0)
                pltpu.sync_copy(v1, dg_s.at[srcb], add=True)
                pltpu.sync_copy(v2, gfd_s.at[srcb], add=True)
            plsc.subcore_barrier()

            pltpu.sync_copy(gf_s.at[pl.ds(nb, _NS)],
                            gf_out.at[pl.ds(i * NPAD + nb, _NS)])
            pltpu.sync_copy(dg_s.at[pl.ds(nb, _NS)],
                            dg_out.at[pl.ds(i * NPAD + nb, _NS)])
            pltpu.sync_copy(gfd_s.at[pl.ds(nb, _NS)],
                            gfd_out.at[pl.ds(i * NPAD + nb, _NS)])


def _curv_stats_call(fT_pad, weights, src, dst, zpad):
    kfn = functools.partial(
        pl.kernel,
        mesh=_mesh(),
        compiler_params=pltpu.CompilerParams(needs_layout_passes=False),
        out_type=(jax.ShapeDtypeStruct((3 * NPAD,), F32),
                  jax.ShapeDtypeStruct((3 * NPAD,), F32),
                  jax.ShapeDtypeStruct((3 * NPAD,), F32)),
        scratch_types=[
            pltpu.VMEM((NPAD,), F32), pltpu.VMEM((NPAD,), F32),
            pltpu.VMEM((NPAD,), F32),
            pltpu.VMEM((_CE,), jnp.int32), pltpu.VMEM((_CE,), jnp.int32),
            pltpu.VMEM((_CE,), F32),
            pltpu.VMEM((_CE,), F32), pltpu.VMEM((_CE,), F32),
            pltpu.VMEM_SHARED((NPAD,), F32), pltpu.VMEM_SHARED((NPAD,), F32),
            pltpu.VMEM_SHARED((NPAD,), F32), pltpu.VMEM_SHARED((NPAD,), F32),
        ],
    )
    return kfn(_curv_stats_kernel)(fT_pad, weights, src, dst, zpad)


_HI_ONE = 1065353216


def _bc(x):
    return jnp.broadcast_to(jnp.asarray(x, jnp.int32), (16,))


def _count_ge(kb, z1):
    z1v = _bc(z1)

    def body(j, a1):
        for u in range(8):
            v = kb[pl.ds((j * 8 + u) * 16, 16)]
            a1 = a1 + (v >= z1v).astype(jnp.int32)
        return a1
    a1 = lax.fori_loop(0, NPAD // 128, body, jnp.zeros((16,), jnp.int32))
    return jnp.sum(a1, axis=0)


def _count_tie_le(kb, t, m):
    tv, mv = _bc(t), _bc(m)
    ii = lax.iota(jnp.int32, 16)

    def body(j, acc):
        for u in range(8):
            o = (j * 8 + u) * 16
            v = kb[pl.ds(o, 16)]
            idx = _bc(o) + ii
            acc = acc + ((v == tv) & (idx <= mv)).astype(jnp.int32)
        return acc
    acc = lax.fori_loop(0, NPAD // 128, body, jnp.zeros((16,), jnp.int32))
    return jnp.sum(acc, axis=0)


def _select_rewrite_kernel(rvec16, kap_pad, src_e, dst_e, d12_out,
                           rvb, kapf, kb, rml, srcb, dstb, outb):
    c = lax.axis_index("c")
    pltpu.sync_copy(rvec16, rvb)
    r = jnp.sum((lax.iota(jnp.int32, 16) == _bc(c)).astype(jnp.int32)
                * rvb[...], axis=0)
    _select_core(r, kap_pad, src_e, dst_e, d12_out,
                 kapf, kb, rml, srcb, dstb, outb, c)


def _select_core(r, kap_pad, src_e, dst_e, d12_out,
                 kapf, kb, rml, srcb, dstb, outb, c):
    s = lax.axis_index("s")
    pltpu.sync_copy(kap_pad, kapf)

    def tobits(j, _):
        for u in range(8):
            o = (j * 8 + u) * 16
            kb[pl.ds(o, 16)] = plsc.bitcast(kapf[pl.ds(o, 16)], jnp.int32)
        return 0
    lax.fori_loop(0, NPAD // 128, tobits, 0)

    def vsearch(_, carry):
        lo, hi = carry
        mid = (lo + hi + 1) >> 1
        big = (_count_ge(kb, mid) >= r).astype(jnp.int32)
        return lo + big * (mid - lo), hi + (1 - big) * (mid - 1 - hi)
    lo, hi = lax.fori_loop(0, 31, vsearch,
                           (jnp.int32(-1), jnp.int32(_HI_ONE)))
    t = lo
    need = r - _count_ge(kb, t + 1)

    def isearch(_, carry):
        lo2, hi2 = carry
        mid = (lo2 + hi2) >> 1
        ok = (_count_tie_le(kb, t, mid) >= need).astype(jnp.int32)
        return (lo2 + (1 - ok) * (mid + 1 - lo2),
                hi2 + ok * (mid - hi2))
    lo2, hi2 = lax.fori_loop(0, 14, isearch,
                             (jnp.int32(0), jnp.int32(N - 1)))
    m = lo2

    tv, mv = _bc(t), _bc(m)
    rposv = _bc((r > 0).astype(jnp.int32))
    ii = lax.iota(jnp.int32, 16)

    def mkmask(j, _):
        for u in range(8):
            o = (j * 8 + u) * 16
            v = kb[pl.ds(o, 16)]
            idx = _bc(o) + ii
            rml[pl.ds(o, 16)] = (
                ((v > tv) | ((v == tv) & (idx <= mv))).astype(jnp.int32)
                * rposv)
        return 0
    lax.fori_loop(0, NPAD // 128, mkmask, 0)

    for k in range(_EPS // _CE):
        off = s * _EPS + k * _CE
        pltpu.sync_copy(src_e.at[pl.ds(off, _CE)], srcb)
        pltpu.sync_copy(dst_e.at[pl.ds(off, _CE)], dstb)

        def rw(j, _):
            for u in range(5):
                o = (j * 5 + u) * 16
                s16 = srcb[pl.ds(o, 16)]
                d16 = dstb[pl.ds(o, 16)]
                drop = (plsc.load_gather(rml, [s16])
                        + plsc.load_gather(rml, [d16])) > 0
                trsh = _bc(NPAD) + ((_bc(off + o) + ii) & _bc(NTRASH - 1))
                outb[pl.ds(o, 16)] = plsc.bitcast(
                    jnp.where(drop, trsh, d16), F32)
            return 0
        lax.fori_loop(0, _CE // 80, rw, 0)

        pltpu.sync_copy(outb, d12_out.at[pl.ds(c * E + off, _CE)])


def _select_rewrite_call(rvec16, kap_pad, src, dst):
    kfn = functools.partial(
        pl.kernel,
        mesh=_mesh(),
        compiler_params=pltpu.CompilerParams(needs_layout_passes=False),
        out_type=jax.ShapeDtypeStruct((2 * E,), F32),
        scratch_types=[
            pltpu.VMEM((16,), jnp.int32),
            pltpu.VMEM((NPAD,), F32), pltpu.VMEM((NPAD,), jnp.int32),
            pltpu.VMEM((NPAD,), jnp.int32),
            pltpu.VMEM((_CE,), jnp.int32), pltpu.VMEM((_CE,), jnp.int32),
            pltpu.VMEM((_CE,), F32),
        ],
    )
    return kfn(_select_rewrite_kernel)(rvec16, kap_pad, src, dst)


_CA = 400
_NSA = NAGG // 16


def _agg_kernel(hq0, hq1, hq2, hq3, src_e, dst_e, zrows,
                a0_out, a1_out, a2_out, a3_out,
                srcb0, srcb1, dstb0, dstb1, rows0, rows1, sem0, sem1, agg_s):
    c = lax.axis_index("c")
    s = lax.axis_index("s")
    rb = s * _NSA
    srcb = (srcb0, srcb1)
    dstb = (dstb0, dstb1)
    rows = (rows0, rows1)
    sem = (sem0, sem1)
    nk = _EPS // _CA
    pltpu.sync_copy(zrows.at[pl.ds(rb, _NSA)], agg_s.at[pl.ds(rb, _NSA)])
    plsc.subcore_barrier()

    for q in range(2):
        hq_c0 = (hq0, hq1)[q]
        hq_c1 = (hq2, hq3)[q]
        out_c0 = (a0_out, a1_out)[q]
        out_c1 = (a2_out, a3_out)[q]

        def fire(k):
            b = k % 2
            off = s * _EPS + k * _CA
            pltpu.sync_copy(src_e.at[pl.ds(off, _CA)], srcb[b])
            pltpu.sync_copy(dst_e.at[pl.ds(off, _CA)], dstb[b])

            @pl.when(c == 0)
            def _():
                pltpu.async_copy(hq_c0.at[srcb[b]], rows[b], sem[b])

            @pl.when(c == 1)
            def _():
                pltpu.async_copy(hq_c1.at[srcb[b]], rows[b], sem[b])

        fire(0)
        for k in range(nk):
            b = k % 2
            if k + 1 < nk:
                fire(k + 1)
            pltpu.make_async_copy(hq_c0.at[srcb[b]], rows[b], sem[b]).wait()
            pltpu.sync_copy(rows[b], agg_s.at[dstb[b]], add=True)

        plsc.subcore_barrier()

        @pl.when(c == 0)
        def _():
            pltpu.sync_copy(agg_s.at[pl.ds(rb, _NSA)],
                            out_c0.at[pl.ds(rb, _NSA)])

        @pl.when(c == 1)
        def _():
            pltpu.sync_copy(agg_s.at[pl.ds(rb, _NSA)],
                            out_c1.at[pl.ds(rb, _NSA)])

        if q == 0:
            pltpu.sync_copy(zrows.at[pl.ds(rb, _NSA)],
                            agg_s.at[pl.ds(rb, _NSA)])
            plsc.subcore_barrier()


def _agg_call(hq0, hq1, hq2, hq3, src, dst_l, zrows):
    kfn = functools.partial(
        pl.kernel,
        mesh=_mesh(),
        compiler_params=pltpu.CompilerParams(use_tc_tiling_on_sc=False),
        out_type=tuple(jax.ShapeDtypeStruct((NAGG, 64), F32)
                       for _ in range(4)),
        scratch_types=[
            pltpu.VMEM((_CA,), jnp.int32), pltpu.VMEM((_CA,), jnp.int32),
            pltpu.VMEM((_CA,), jnp.int32), pltpu.VMEM((_CA,), jnp.int32),
            pltpu.VMEM((_CA, 64), F32), pltpu.VMEM((_CA, 64), F32),
            pltpu.SemaphoreType.DMA, pltpu.SemaphoreType.DMA,
            pltpu.VMEM_SHARED((NAGG, 64), F32),
        ],
    )
    return kfn(_agg_kernel)(hq0, hq1, hq2, hq3, src, dst_l, zrows)


def kernel(h, x, edge_index, edge_attr, norm_edge_weight, batch_idx, p,
           We_h, be_h, Wc1, bc1, Wc2, bc2, Wf1, bf1, Wf2, bf2,
           Wm1, bm1, Wm2, bm2, Wm3, bm3, Wg, bg, bn_gamma, bn_beta,
           Wlp, blp, Wfc2, bfc2):
    src = edge_index[0]
    dst = edge_index[1]

    hh = _hh_call(h, We_h, be_h.reshape(1, F))
    W2p = jnp.zeros((128, 20), F32)
    W2p = W2p.at[0:1].set(Wc2.reshape(1, 20))
    W2p = W2p.at[1:4].set(Wf2.reshape(3, 20))
    kp, fp0, fp1, fp2 = _maps_call(
        x, Wc1.reshape(1, 20), bc1.reshape(1, 20),
        W2p, bc2.reshape(1, 1),
        Wf1.reshape(3, 1, 20), bf1.reshape(3, 1, 20), bf2.reshape(3, 1, 1))
    kap2d = jax.nn.sigmoid(kp)
    f0, f1, f2 = (jax.nn.sigmoid(fp0), jax.nn.sigmoid(fp1),
                  jax.nn.sigmoid(fp2))

    m1 = _wm1_call(edge_attr.reshape(1, E), Wm1, bm1.reshape(64, 1))
    w2 = _w2_call(m1, Wm2, bm2.reshape(64, 1))
    weights = _wrow_call(w2.reshape(1, 64), Wm3, bm3.reshape(1, E)).reshape(E)

    fT_pad = jnp.concatenate(
        [jnp.pad(fi.reshape(N), (0, NPAD - N)) for fi in (f0, f1, f2)])
    zpad = jnp.zeros((NPAD,), F32)
    gf, dg, gfd = _curv_stats_call(fT_pad, weights, src, dst, zpad)
    gf = gf.reshape(3, NPAD)
    dg = dg.reshape(3, NPAD)
    gfd = gfd.reshape(3, NPAD)

    curv = _curv_call(kap2d, gf, dg, gfd).reshape(())

    p32 = jnp.asarray(p, jnp.int32)
    r1 = (N * p32 * 1) // 100
    r2 = (N * p32 * 2) // 100
    rvec16 = jnp.zeros((16,), jnp.int32).at[0].set(r1).at[1].set(r2)
    kap_pad = jnp.pad(kap2d.reshape(N), (0, NPAD - N), constant_values=-1.0)
    d12 = lax.bitcast_convert_type(
        _select_rewrite_call(rvec16, kap_pad, src, dst), jnp.int32)
    dst_by_layer = [dst, d12[:E], d12[E:]]

    zrows = jnp.zeros((NAGG, 64), F32)
    hcur = hh
    for layer in range(3):
        a0, a1, a2, a3 = _agg_call(hcur[:, 0:64], hcur[:, 64:128],
                                   hcur[:, 128:192], hcur[:, 192:256],
                                   src, dst_by_layer[layer], zrows)
        y, sm, ssq = _stats_call(hcur, a0, a1, a2, a3,
                                 Wg[layer], bg[layer].reshape(1, F))
        hcur = _finish_call(y, hcur, sm, ssq,
                            bn_gamma[layer].reshape(1, F),
                            bn_beta[layer].reshape(1, F))

    ho = _pool_call(hcur, batch_idx.reshape(N // RB, 1, RB).astype(F32),
                    Wlp, blp.reshape(1, 2 * F), Wfc2.reshape(1, 2 * F),
                    bfc2.reshape(1, 1))
    return ho, curv
